# Initial kernel scaffold; baseline (speedup 1.0000x reference)
import jax
import jax.numpy as jnp
from jax.experimental import pallas as pl


def kernel(v, edge_index, W1, b1, W2, b2):
    raise NotImplementedError("write your pallas kernel here")


# speedup vs baseline: 1.9853x; 1.9853x over previous
import functools

import jax
import jax.numpy as jnp
from jax import lax
from jax.experimental import pallas as pl
from jax.experimental.pallas import tpu as pltpu
from jax.experimental.pallas import tpu_sc as plsc

N_NODES = 10000
N_EDGES = 320000
N_OUT = 128
HIDDEN = 384

NC = 2
NS = 16
CHUNK = 128
CHUNKS_PER_W = 80
E_PAD = NC * NS * CHUNKS_PER_W * CHUNK
N_ACC = 10112
ROWS_PER_TILE = N_ACC // NS
BE = 2048
DUMMY = N_NODES


def _mlp_body(v_ref, w1_ref, b1_ref, w2_ref, b2_ref, o_ref):
    h = jnp.dot(v_ref[...], w1_ref[...], preferred_element_type=jnp.float32)
    h = jnp.maximum(h + b1_ref[...], 0.0)
    o_ref[...] = (
        jnp.dot(h, w2_ref[...], preferred_element_type=jnp.float32)
        + b2_ref[...]
    )


def _mlp(v_pad, W1, b1, W2, b2):
    grid = E_PAD // BE
    return pl.pallas_call(
        _mlp_body,
        grid=(grid,),
        in_specs=[
            pl.BlockSpec((BE, 2), lambda i: (i, 0)),
            pl.BlockSpec((2, HIDDEN), lambda i: (0, 0)),
            pl.BlockSpec((1, HIDDEN), lambda i: (0, 0)),
            pl.BlockSpec((HIDDEN, N_OUT), lambda i: (0, 0)),
            pl.BlockSpec((1, N_OUT), lambda i: (0, 0)),
        ],
        out_specs=pl.BlockSpec((BE, N_OUT), lambda i: (i, 0)),
        out_shape=jax.ShapeDtypeStruct((E_PAD, N_OUT), jnp.float32),
    )(v_pad, W1, b1, W2, b2)


def _scatter_body(s_hbm, col_hbm, zeros_hbm, out_hbm, idx_v, rows_v, acc):
    cid = lax.axis_index("c")
    sid = lax.axis_index("s")
    wid = cid * NS + sid

    r0 = sid * ROWS_PER_TILE
    pltpu.sync_copy(zeros_hbm.at[pl.ds(r0, ROWS_PER_TILE)],
                    acc.at[pl.ds(r0, ROWS_PER_TILE)])

    c0 = wid * CHUNKS_PER_W
    pltpu.sync_copy(col_hbm.at[pl.ds(c0, CHUNKS_PER_W)], idx_v)
    plsc.subcore_barrier()

    base = wid * (CHUNKS_PER_W * CHUNK)

    def step(j, carry):
        pltpu.sync_copy(s_hbm.at[pl.ds(base + j * CHUNK, CHUNK)], rows_v)
        pltpu.sync_copy(rows_v, acc.at[idx_v.at[j]], add=True)
        return carry

    lax.fori_loop(0, CHUNKS_PER_W, step, 0)
    plsc.subcore_barrier()

    pltpu.sync_copy(acc.at[pl.ds(r0, ROWS_PER_TILE)],
                    out_hbm.at[cid, pl.ds(r0, ROWS_PER_TILE)])


def _scatter(s, col_pad, zeros):
    mesh = plsc.VectorSubcoreMesh(core_axis_name="c", subcore_axis_name="s")
    f = pl.kernel(
        _scatter_body,
        out_type=jax.ShapeDtypeStruct((NC, N_ACC, N_OUT), jnp.float32),
        mesh=mesh,
        scratch_types=[
            pltpu.VMEM((CHUNKS_PER_W, CHUNK), jnp.int32),
            pltpu.VMEM((CHUNK, N_OUT), jnp.float32),
            pltpu.VMEM_SHARED((N_ACC, N_OUT), jnp.float32),
        ],
    )
    return f(s, col_pad, zeros)


def _combine_body(p_ref, o_ref):
    o_ref[...] = p_ref[0] + p_ref[1]


def _combine(partials):
    return pl.pallas_call(
        _combine_body,
        out_shape=jax.ShapeDtypeStruct((N_ACC, N_OUT), jnp.float32),
    )(partials)


def kernel(v, edge_index, W1, b1, W2, b2):
    col = edge_index[1].astype(jnp.int32)
    col_pad = jnp.concatenate(
        [col, jnp.full((E_PAD - N_EDGES,), DUMMY, jnp.int32)]
    ).reshape(E_PAD // CHUNK, CHUNK)
    v_pad = jnp.pad(v, ((0, E_PAD - N_EDGES), (0, 0)))
    s = _mlp(v_pad, W1, b1.reshape(1, -1), W2, b2.reshape(1, -1))
    zeros = jnp.zeros((N_ACC, N_OUT), jnp.float32)
    partials = _scatter(s, col_pad, zeros)
    out = _combine(partials)
    return out[:N_NODES]


# speedup vs baseline: 2.2301x; 1.1233x over previous
import functools

import jax
import jax.numpy as jnp
from jax import lax
from jax.experimental import pallas as pl
from jax.experimental.pallas import tpu as pltpu
from jax.experimental.pallas import tpu_sc as plsc

N_NODES = 10000
N_EDGES = 320000
N_OUT = 128
HIDDEN = 384

NC = 2
NS = 16
CHUNK = 128
CHUNKS_PER_W = 80
E_PAD = NC * NS * CHUNKS_PER_W * CHUNK
N_ACC = 10112
ROWS_PER_TILE = N_ACC // NS
BE = 2048
DUMMY = N_NODES


def _mlp_body(v_ref, w1_ref, b1_ref, w2_ref, b2_ref, o_ref):
    h = jnp.dot(v_ref[...], w1_ref[...], preferred_element_type=jnp.float32)
    h = jnp.maximum(h + b1_ref[...], 0.0)
    o_ref[...] = (
        jnp.dot(h, w2_ref[...], preferred_element_type=jnp.float32)
        + b2_ref[...]
    )


def _mlp(v_pad, W1, b1, W2, b2):
    grid = E_PAD // BE
    return pl.pallas_call(
        _mlp_body,
        grid=(grid,),
        in_specs=[
            pl.BlockSpec((BE, 2), lambda i: (i, 0)),
            pl.BlockSpec((2, HIDDEN), lambda i: (0, 0)),
            pl.BlockSpec((1, HIDDEN), lambda i: (0, 0)),
            pl.BlockSpec((HIDDEN, N_OUT), lambda i: (0, 0)),
            pl.BlockSpec((1, N_OUT), lambda i: (0, 0)),
        ],
        out_specs=pl.BlockSpec((BE, N_OUT), lambda i: (i, 0)),
        out_shape=jax.ShapeDtypeStruct((E_PAD, N_OUT), jnp.float32),
    )(v_pad, W1, b1, W2, b2)


def _scatter_body(s_hbm, col_hbm, zeros_hbm, out_hbm, idx_v, rows0, rows1,
                  acc, sem0, sem1):
    cid = lax.axis_index("c")
    sid = lax.axis_index("s")
    wid = cid * NS + sid

    r0 = sid * ROWS_PER_TILE
    pltpu.sync_copy(zeros_hbm.at[pl.ds(r0, ROWS_PER_TILE)],
                    acc.at[pl.ds(r0, ROWS_PER_TILE)])

    c0 = wid * CHUNKS_PER_W
    pltpu.sync_copy(col_hbm.at[pl.ds(c0, CHUNKS_PER_W)], idx_v)
    plsc.subcore_barrier()

    base = wid * (CHUNKS_PER_W * CHUNK)

    def src(j):
        return s_hbm.at[pl.ds(base + (j % CHUNKS_PER_W) * CHUNK, CHUNK)]

    pltpu.async_copy(src(0), rows0, sem0)
    pltpu.async_copy(src(1), rows1, sem1)

    def step(i, carry):
        j0 = i * 2
        pltpu.make_async_copy(src(j0), rows0, sem0).wait()
        pltpu.sync_copy(rows0, acc.at[idx_v.at[j0]], add=True)
        pltpu.async_copy(src(j0 + 2), rows0, sem0)
        pltpu.make_async_copy(src(j0 + 1), rows1, sem1).wait()
        pltpu.sync_copy(rows1, acc.at[idx_v.at[j0 + 1]], add=True)
        pltpu.async_copy(src(j0 + 3), rows1, sem1)
        return carry

    lax.fori_loop(0, CHUNKS_PER_W // 2, step, 0)
    pltpu.make_async_copy(src(0), rows0, sem0).wait()
    pltpu.make_async_copy(src(1), rows1, sem1).wait()
    plsc.subcore_barrier()

    pltpu.sync_copy(acc.at[pl.ds(r0, ROWS_PER_TILE)],
                    out_hbm.at[cid, pl.ds(r0, ROWS_PER_TILE)])


def _scatter(s, col_pad, zeros):
    mesh = plsc.VectorSubcoreMesh(core_axis_name="c", subcore_axis_name="s")
    f = pl.kernel(
        _scatter_body,
        out_type=jax.ShapeDtypeStruct((NC, N_ACC, N_OUT), jnp.float32),
        mesh=mesh,
        scratch_types=[
            pltpu.VMEM((CHUNKS_PER_W, CHUNK), jnp.int32),
            pltpu.VMEM((CHUNK, N_OUT), jnp.float32),
            pltpu.VMEM((CHUNK, N_OUT), jnp.float32),
            pltpu.VMEM_SHARED((N_ACC, N_OUT), jnp.float32),
            pltpu.SemaphoreType.DMA,
            pltpu.SemaphoreType.DMA,
        ],
    )
    return f(s, col_pad, zeros)


def _combine_body(p_ref, o_ref):
    o_ref[...] = p_ref[0] + p_ref[1]


def _combine(partials):
    return pl.pallas_call(
        _combine_body,
        out_shape=jax.ShapeDtypeStruct((N_ACC, N_OUT), jnp.float32),
    )(partials)


def kernel(v, edge_index, W1, b1, W2, b2):
    col = edge_index[1].astype(jnp.int32)
    col_pad = jnp.concatenate(
        [col, jnp.full((E_PAD - N_EDGES,), DUMMY, jnp.int32)]
    ).reshape(E_PAD // CHUNK, CHUNK)
    v_pad = jnp.pad(v, ((0, E_PAD - N_EDGES), (0, 0)))
    s = _mlp(v_pad, W1, b1.reshape(1, -1), W2, b2.reshape(1, -1))
    zeros = jnp.zeros((N_ACC, N_OUT), jnp.float32)
    partials = _scatter(s, col_pad, zeros)
    out = _combine(partials)
    return out[:N_NODES]


# speedup vs baseline: 2.7860x; 1.2493x over previous
import functools

import jax
import jax.numpy as jnp
from jax import lax
from jax.experimental import pallas as pl
from jax.experimental.pallas import tpu as pltpu
from jax.experimental.pallas import tpu_sc as plsc

N_NODES = 10000
N_EDGES = 320000
N_OUT = 128
HIDDEN = 384

NC = 2
NS = 16
CHUNK = 128
CHUNKS_PER_W = 80
E_PAD = NC * NS * CHUNKS_PER_W * CHUNK
N_ACC = 10112
ROWS_PER_TILE = N_ACC // NS
BE = 2048
DUMMY = N_NODES


def _mlp_body(v_ref, w1_ref, b1_ref, w2_ref, b2_ref, o_ref):
    h = jnp.dot(v_ref[...], w1_ref[...], preferred_element_type=jnp.float32)
    h = jnp.maximum(h + b1_ref[...], 0.0)
    o_ref[...] = (
        jnp.dot(h, w2_ref[...], preferred_element_type=jnp.float32)
        + b2_ref[...]
    )


def _mlp(v, W1, b1, W2, b2):
    grid = E_PAD // BE
    last = N_EDGES // BE
    return pl.pallas_call(
        _mlp_body,
        grid=(grid,),
        in_specs=[
            pl.BlockSpec((BE, 2), lambda i: (jnp.minimum(i, last), 0)),
            pl.BlockSpec((2, HIDDEN), lambda i: (0, 0)),
            pl.BlockSpec((1, HIDDEN), lambda i: (0, 0)),
            pl.BlockSpec((HIDDEN, N_OUT), lambda i: (0, 0)),
            pl.BlockSpec((1, N_OUT), lambda i: (0, 0)),
        ],
        out_specs=pl.BlockSpec((BE, N_OUT), lambda i: (i, 0)),
        out_shape=jax.ShapeDtypeStruct((E_PAD, N_OUT), jnp.float32),
    )(v, W1, b1, W2, b2)


def _scatter_body(s_hbm, col_hbm, zeros_hbm, out_hbm, idx_v, rows0, rows1,
                  acc, sem0, sem1):
    cid = lax.axis_index("c")
    sid = lax.axis_index("s")
    wid = cid * NS + sid

    r0 = sid * ROWS_PER_TILE
    pltpu.sync_copy(zeros_hbm.at[pl.ds(r0, ROWS_PER_TILE)],
                    acc.at[pl.ds(r0, ROWS_PER_TILE)])

    c0 = wid * CHUNKS_PER_W
    pltpu.sync_copy(col_hbm.at[pl.ds(c0, CHUNKS_PER_W)], idx_v)
    plsc.subcore_barrier()

    base = wid * (CHUNKS_PER_W * CHUNK)

    def src(j):
        return s_hbm.at[pl.ds(base + (j % CHUNKS_PER_W) * CHUNK, CHUNK)]

    pltpu.async_copy(src(0), rows0, sem0)
    pltpu.async_copy(src(1), rows1, sem1)

    def step(i, carry):
        j0 = i * 2
        pltpu.make_async_copy(src(j0), rows0, sem0).wait()
        pltpu.sync_copy(rows0, acc.at[idx_v.at[j0]], add=True)
        pltpu.async_copy(src(j0 + 2), rows0, sem0)
        pltpu.make_async_copy(src(j0 + 1), rows1, sem1).wait()
        pltpu.sync_copy(rows1, acc.at[idx_v.at[j0 + 1]], add=True)
        pltpu.async_copy(src(j0 + 3), rows1, sem1)
        return carry

    lax.fori_loop(0, CHUNKS_PER_W // 2, step, 0)
    pltpu.make_async_copy(src(0), rows0, sem0).wait()
    pltpu.make_async_copy(src(1), rows1, sem1).wait()
    plsc.subcore_barrier()

    pltpu.sync_copy(acc.at[pl.ds(r0, ROWS_PER_TILE)],
                    out_hbm.at[cid, pl.ds(r0, ROWS_PER_TILE)])


def _scatter(s, col_pad, zeros):
    mesh = plsc.VectorSubcoreMesh(core_axis_name="c", subcore_axis_name="s")
    f = pl.kernel(
        _scatter_body,
        out_type=jax.ShapeDtypeStruct((NC, N_ACC, N_OUT), jnp.float32),
        mesh=mesh,
        scratch_types=[
            pltpu.VMEM((CHUNKS_PER_W, CHUNK), jnp.int32),
            pltpu.VMEM((CHUNK, N_OUT), jnp.float32),
            pltpu.VMEM((CHUNK, N_OUT), jnp.float32),
            pltpu.VMEM_SHARED((N_ACC, N_OUT), jnp.float32),
            pltpu.SemaphoreType.DMA,
            pltpu.SemaphoreType.DMA,
        ],
    )
    return f(s, col_pad, zeros)


def _combine_body(p0_ref, p1_ref, o_ref):
    o_ref[...] = p0_ref[0] + p1_ref[0]


def _combine(partials):
    blk = 2000
    return pl.pallas_call(
        _combine_body,
        grid=(N_NODES // blk,),
        in_specs=[
            pl.BlockSpec((1, blk, N_OUT), lambda i: (0, i, 0)),
            pl.BlockSpec((1, blk, N_OUT), lambda i: (1, i, 0)),
        ],
        out_specs=pl.BlockSpec((blk, N_OUT), lambda i: (i, 0)),
        out_shape=jax.ShapeDtypeStruct((N_NODES, N_OUT), jnp.float32),
    )(partials, partials)


def kernel(v, edge_index, W1, b1, W2, b2):
    col = edge_index[1].astype(jnp.int32)
    col_pad = jnp.concatenate(
        [col, jnp.full((E_PAD - N_EDGES,), DUMMY, jnp.int32)]
    ).reshape(E_PAD // CHUNK, CHUNK)
    s = _mlp(v, W1, b1.reshape(1, -1), W2, b2.reshape(1, -1))
    zeros = jnp.zeros((N_ACC, N_OUT), jnp.float32)
    partials = _scatter(s, col_pad, zeros)
    return _combine(partials)


# speedup vs baseline: 3.7072x; 1.3307x over previous
import functools

import jax
import jax.numpy as jnp
from jax import lax
from jax.experimental import pallas as pl
from jax.experimental.pallas import tpu as pltpu
from jax.experimental.pallas import tpu_sc as plsc

N_NODES = 10000
N_EDGES = 320000
N_OUT = 128
HIDDEN = 384

NC = 2
NS = 16
CHUNK = 128
CHUNKS_PER_W = 80
E_PAD = NC * NS * CHUNKS_PER_W * CHUNK
N_ACC = 10112
ROWS_PER_TILE = N_ACC // NS
BE = 2048
DUMMY = N_NODES


def _mlp_body(vt_ref, w1_ref, b1_ref, w2_ref, b2_ref, o_ref):
    h = lax.dot_general(vt_ref[...], w1_ref[...],
                        (((0,), (0,)), ((), ())),
                        preferred_element_type=jnp.float32)
    h = jnp.maximum(h + b1_ref[...], 0.0)
    o_ref[...] = (
        jnp.dot(h, w2_ref[...], preferred_element_type=jnp.float32)
        + b2_ref[...]
    )


def _mlp(vt, W1, b1, W2, b2):
    grid = E_PAD // BE
    last = N_EDGES // BE
    return pl.pallas_call(
        _mlp_body,
        grid=(grid,),
        in_specs=[
            pl.BlockSpec((2, BE), lambda i: (0, jnp.minimum(i, last))),
            pl.BlockSpec((2, HIDDEN), lambda i: (0, 0)),
            pl.BlockSpec((1, HIDDEN), lambda i: (0, 0)),
            pl.BlockSpec((HIDDEN, N_OUT), lambda i: (0, 0)),
            pl.BlockSpec((1, N_OUT), lambda i: (0, 0)),
        ],
        out_specs=pl.BlockSpec((BE, N_OUT), lambda i: (i, 0)),
        out_shape=jax.ShapeDtypeStruct((E_PAD, N_OUT), jnp.float32),
    )(vt, W1, b1, W2, b2)


def _scatter_body(s_hbm, col_hbm, zeros_hbm, out_hbm, idx_v, rows0, rows1,
                  acc, sem0, sem1):
    cid = lax.axis_index("c")
    sid = lax.axis_index("s")
    wid = cid * NS + sid

    r0 = sid * ROWS_PER_TILE
    pltpu.sync_copy(zeros_hbm.at[pl.ds(r0, ROWS_PER_TILE)],
                    acc.at[pl.ds(r0, ROWS_PER_TILE)])

    c0 = wid * CHUNKS_PER_W
    pltpu.sync_copy(col_hbm.at[pl.ds(c0, CHUNKS_PER_W)], idx_v)
    plsc.subcore_barrier()

    base = wid * (CHUNKS_PER_W * CHUNK)

    def src(j):
        return s_hbm.at[pl.ds(base + (j % CHUNKS_PER_W) * CHUNK, CHUNK)]

    pltpu.async_copy(src(0), rows0, sem0)
    pltpu.async_copy(src(1), rows1, sem1)

    def step(i, carry):
        j0 = i * 2
        pltpu.make_async_copy(src(j0), rows0, sem0).wait()
        pltpu.sync_copy(rows0, acc.at[idx_v.at[j0]], add=True)
        pltpu.async_copy(src(j0 + 2), rows0, sem0)
        pltpu.make_async_copy(src(j0 + 1), rows1, sem1).wait()
        pltpu.sync_copy(rows1, acc.at[idx_v.at[j0 + 1]], add=True)
        pltpu.async_copy(src(j0 + 3), rows1, sem1)
        return carry

    lax.fori_loop(0, CHUNKS_PER_W // 2, step, 0)
    pltpu.make_async_copy(src(0), rows0, sem0).wait()
    pltpu.make_async_copy(src(1), rows1, sem1).wait()
    plsc.subcore_barrier()

    pltpu.sync_copy(acc.at[pl.ds(r0, ROWS_PER_TILE)],
                    out_hbm.at[cid, pl.ds(r0, ROWS_PER_TILE)])


def _scatter(s, col_pad, zeros):
    mesh = plsc.VectorSubcoreMesh(core_axis_name="c", subcore_axis_name="s")
    f = pl.kernel(
        _scatter_body,
        out_type=jax.ShapeDtypeStruct((NC, N_ACC, N_OUT), jnp.float32),
        mesh=mesh,
        scratch_types=[
            pltpu.VMEM((CHUNKS_PER_W, CHUNK), jnp.int32),
            pltpu.VMEM((CHUNK, N_OUT), jnp.float32),
            pltpu.VMEM((CHUNK, N_OUT), jnp.float32),
            pltpu.VMEM_SHARED((N_ACC, N_OUT), jnp.float32),
            pltpu.SemaphoreType.DMA,
            pltpu.SemaphoreType.DMA,
        ],
    )
    return f(s, col_pad, zeros)


def _combine_body(p0_ref, p1_ref, o_ref):
    o_ref[...] = p0_ref[0] + p1_ref[0]


def _combine(partials):
    blk = 2000
    return pl.pallas_call(
        _combine_body,
        grid=(N_NODES // blk,),
        in_specs=[
            pl.BlockSpec((1, blk, N_OUT), lambda i: (0, i, 0)),
            pl.BlockSpec((1, blk, N_OUT), lambda i: (1, i, 0)),
        ],
        out_specs=pl.BlockSpec((blk, N_OUT), lambda i: (i, 0)),
        out_shape=jax.ShapeDtypeStruct((N_NODES, N_OUT), jnp.float32),
    )(partials, partials)


def kernel(v, edge_index, W1, b1, W2, b2):
    col = edge_index[1].astype(jnp.int32)
    col_pad = jnp.concatenate(
        [col, jnp.full((E_PAD - N_EDGES,), DUMMY, jnp.int32)]
    ).reshape(E_PAD // CHUNK, CHUNK)
    s = _mlp(v.T, W1, b1.reshape(1, -1), W2, b2.reshape(1, -1))
    zeros = jnp.zeros((N_ACC, N_OUT), jnp.float32)
    partials = _scatter(s, col_pad, zeros)
    return _combine(partials)


# speedup vs baseline: 4.0391x; 1.0895x over previous
import jax
import jax.numpy as jnp
from jax import lax
from jax.experimental import pallas as pl
from jax.experimental.pallas import tpu as pltpu
from jax.experimental.pallas import tpu_sc as plsc

N_NODES = 10000
N_EDGES = 320000
N_OUT = 128
HIDDEN = 384

NC = 2
NS = 16
CHUNK = 128
E_HALF = N_EDGES // 2
CHUNKS_PER_W = 40
E_PAD = NC * NS * CHUNKS_PER_W * CHUNK
N_ACC = 10112
ROWS_PER_TILE = N_ACC // NS
BE = 2048
DUMMY = N_NODES


def _mlp_body(vt_ref, w1_ref, b1_ref, w2_ref, b2_ref, o_ref):
    h = lax.dot_general(vt_ref[...], w1_ref[...],
                        (((0,), (0,)), ((), ())),
                        preferred_element_type=jnp.float32)
    h = jnp.maximum(h + b1_ref[...], 0.0)
    o_ref[...] = (
        jnp.dot(h, w2_ref[...], preferred_element_type=jnp.float32)
        + b2_ref[...]
    )


def _mlp(vt_half, W1, b1, W2, b2):
    grid = E_PAD // BE
    last = E_HALF // BE
    return pl.pallas_call(
        _mlp_body,
        grid=(grid,),
        in_specs=[
            pl.BlockSpec((2, BE), lambda i: (0, jnp.minimum(i, last))),
            pl.BlockSpec((2, HIDDEN), lambda i: (0, 0)),
            pl.BlockSpec((1, HIDDEN), lambda i: (0, 0)),
            pl.BlockSpec((HIDDEN, N_OUT), lambda i: (0, 0)),
            pl.BlockSpec((1, N_OUT), lambda i: (0, 0)),
        ],
        out_specs=pl.BlockSpec((BE, N_OUT), lambda i: (i, 0)),
        out_shape=jax.ShapeDtypeStruct((E_PAD, N_OUT), jnp.float32),
    )(vt_half, W1, b1, W2, b2)


def _scatter_body(s_hbm, col_hbm, zeros_hbm, out_hbm, idx_v, rows0, rows1,
                  acc, sem0, sem1):
    cid = lax.axis_index("c")
    sid = lax.axis_index("s")
    wid = cid * NS + sid

    r0 = sid * ROWS_PER_TILE
    pltpu.sync_copy(zeros_hbm.at[pl.ds(r0, ROWS_PER_TILE)],
                    acc.at[pl.ds(r0, ROWS_PER_TILE)])

    c0 = wid * CHUNKS_PER_W
    pltpu.sync_copy(col_hbm.at[pl.ds(c0, CHUNKS_PER_W)], idx_v)
    plsc.subcore_barrier()

    base = wid * (CHUNKS_PER_W * CHUNK)

    def src(j):
        return s_hbm.at[pl.ds(base + (j % CHUNKS_PER_W) * CHUNK, CHUNK)]

    pltpu.async_copy(src(0), rows0, sem0)
    pltpu.async_copy(src(1), rows1, sem1)

    def step(i, carry):
        j0 = i * 2
        pltpu.make_async_copy(src(j0), rows0, sem0).wait()
        pltpu.sync_copy(rows0, acc.at[idx_v.at[j0]], add=True)
        pltpu.async_copy(src(j0 + 2), rows0, sem0)
        pltpu.make_async_copy(src(j0 + 1), rows1, sem1).wait()
        pltpu.sync_copy(rows1, acc.at[idx_v.at[j0 + 1]], add=True)
        pltpu.async_copy(src(j0 + 3), rows1, sem1)
        return carry

    lax.fori_loop(0, CHUNKS_PER_W // 2, step, 0)
    pltpu.make_async_copy(src(0), rows0, sem0).wait()
    pltpu.make_async_copy(src(1), rows1, sem1).wait()
    plsc.subcore_barrier()

    pltpu.sync_copy(acc.at[pl.ds(r0, ROWS_PER_TILE)],
                    out_hbm.at[cid, pl.ds(r0, ROWS_PER_TILE)])


def _scatter(s, col_pad, zeros):
    mesh = plsc.VectorSubcoreMesh(core_axis_name="c", subcore_axis_name="s")
    f = pl.kernel(
        _scatter_body,
        out_type=jax.ShapeDtypeStruct((NC, N_ACC, N_OUT), jnp.float32),
        mesh=mesh,
        scratch_types=[
            pltpu.VMEM((CHUNKS_PER_W, CHUNK), jnp.int32),
            pltpu.VMEM((CHUNK, N_OUT), jnp.float32),
            pltpu.VMEM((CHUNK, N_OUT), jnp.float32),
            pltpu.VMEM_SHARED((N_ACC, N_OUT), jnp.float32),
            pltpu.SemaphoreType.DMA,
            pltpu.SemaphoreType.DMA,
        ],
    )
    return f(s, col_pad, zeros)


def _combine_body(a0_ref, a1_ref, b0_ref, b1_ref, o_ref):
    o_ref[...] = (a0_ref[0] + a1_ref[0]) + (b0_ref[0] + b1_ref[0])


def _combine(pa, pb):
    blk = 2000
    spec = [pl.BlockSpec((1, blk, N_OUT), lambda i: (0, i, 0)),
            pl.BlockSpec((1, blk, N_OUT), lambda i: (1, i, 0))]
    return pl.pallas_call(
        _combine_body,
        grid=(N_NODES // blk,),
        in_specs=spec + spec,
        out_specs=pl.BlockSpec((blk, N_OUT), lambda i: (i, 0)),
        out_shape=jax.ShapeDtypeStruct((N_NODES, N_OUT), jnp.float32),
    )(pa, pa, pb, pb)


def _col_half(col_half):
    return jnp.concatenate(
        [col_half, jnp.full((E_PAD - E_HALF,), DUMMY, jnp.int32)]
    ).reshape(E_PAD // CHUNK, CHUNK)


def kernel(v, edge_index, W1, b1, W2, b2):
    col = edge_index[1].astype(jnp.int32)
    vt = v.T
    b1r = b1.reshape(1, -1)
    b2r = b2.reshape(1, -1)
    zeros = jnp.zeros((N_ACC, N_OUT), jnp.float32)

    vt_a = lax.slice(vt, (0, 0), (2, E_HALF))
    vt_b = lax.slice(vt, (0, E_HALF), (2, N_EDGES))
    col_a = _col_half(lax.slice(col, (0,), (E_HALF,)))
    col_b = _col_half(lax.slice(col, (E_HALF,), (N_EDGES,)))

    s_a = _mlp(vt_a, W1, b1r, W2, b2r)
    p_a = _scatter(s_a, col_a, zeros)
    s_b = _mlp(vt_b, W1, b1r, W2, b2r)
    p_b = _scatter(s_b, col_b, zeros)
    return _combine(p_a, p_b)
